# baseline (speedup 1.0000x reference)
import functools

import jax
import jax.numpy as jnp
from jax import lax
from jax.experimental import pallas as pl
from jax.experimental.pallas import tpu as pltpu
from jax.experimental.pallas import tpu_sc as plsc

N_NODES = 50000
N_EDGES = 800000
F = 64

_NODE_BLK = 2000


def _node_transform_body(x_ref, wcat_ref, xcat_ref):
    xcat_ref[...] = jnp.dot(x_ref[...], wcat_ref[...],
                            preferred_element_type=jnp.float32)


def _node_transform(x, wcat):
    n_blk = N_NODES // _NODE_BLK
    return pl.pallas_call(
        _node_transform_body,
        grid=(n_blk,),
        in_specs=[
            pl.BlockSpec((_NODE_BLK, F), lambda i: (i, 0)),
            pl.BlockSpec((F, 2 * F), lambda i: (0, 0)),
        ],
        out_specs=pl.BlockSpec((_NODE_BLK, 2 * F), lambda i: (i, 0)),
        out_shape=jax.ShapeDtypeStruct((N_NODES, 2 * F), jnp.float32),
    )(x, wcat)


_NW = 32
_PER_W = N_EDGES // _NW
_CH = 128


def _gather_body(e0, n_edges, xcat_hbm, idxn_hbm, idxp_hbm, gsum_hbm,
                 idxn_v0, idxn_v1, idxp_v0, idxp_v1,
                 rows_n0, rows_n1, rows_p0, rows_p1, gsum_v0, gsum_v1,
                 sem_i0, sem_i1, sem_g0, sem_g1, sem_s0, sem_s1):
    wid = lax.axis_index("s") * 2 + lax.axis_index("c")
    nchunks = n_edges // _CH
    nch_base = nchunks // _NW
    nch_rem = nchunks - nch_base * _NW
    n_ch = nch_base + jnp.where(wid < nch_rem, 1, 0)
    loc_of = lambda j: (wid + _NW * j) * _CH
    off_of = lambda j: e0 + loc_of(j)
    idxn = (idxn_v0, idxn_v1)
    idxp = (idxp_v0, idxp_v1)
    rows_n = (rows_n0, rows_n1)
    rows_p = (rows_p0, rows_p1)
    sem_i = (sem_i0, sem_i1)
    sem_g = (sem_g0, sem_g1)
    gsum = (gsum_v0, gsum_v1)
    sem_s = (sem_s0, sem_s1)

    def issue_idx(i, s):
        off = off_of(i)
        pltpu.async_copy(idxn_hbm.at[pl.ds(off, _CH)], idxn[s], sem_i[s])
        pltpu.async_copy(idxp_hbm.at[pl.ds(off, _CH)], idxp[s], sem_i[s])

    def wait_idx(i, s):
        off = off_of(i)
        pltpu.make_async_copy(idxn_hbm.at[pl.ds(off, _CH)], idxn[s], sem_i[s]).wait()
        pltpu.make_async_copy(idxp_hbm.at[pl.ds(off, _CH)], idxp[s], sem_i[s]).wait()

    def issue_gather(s):
        pltpu.async_copy(xcat_hbm.at[idxn[s]], rows_n[s], sem_g[s])
        pltpu.async_copy(xcat_hbm.at[idxp[s]], rows_p[s], sem_g[s])

    def wait_gather(s):
        pltpu.make_async_copy(xcat_hbm.at[idxn[s]], rows_n[s], sem_g[s]).wait()
        pltpu.make_async_copy(xcat_hbm.at[idxp[s]], rows_p[s], sem_g[s]).wait()

    issue_idx(0, 0)
    wait_idx(0, 0)
    issue_gather(0)
    issue_idx(1, 1)

    def body(i, s):
        q = 1 - s

        @pl.when(i + 1 < n_ch)
        def _():
            wait_idx(i + 1, q)
            issue_gather(q)

        wait_gather(s)

        @pl.when(i + 2 < n_ch)
        def _():
            issue_idx(i + 2, s)

        @pl.when(i >= 2)
        def _():
            off2 = loc_of(i - 2)
            pltpu.make_async_copy(gsum[s], gsum_hbm.at[pl.ds(off2, _CH)],
                                  sem_s[s]).wait()

        def row(r, c):
            for k in range(F // 16):
                a = rows_n[s][r, pl.ds(16 * k, 16)]
                b = rows_p[s][r, pl.ds(F + 16 * k, 16)]
                gsum[s][r, pl.ds(16 * k, 16)] = a + b
            return c

        lax.fori_loop(0, _CH, row, 0)
        pltpu.async_copy(gsum[s], gsum_hbm.at[pl.ds(loc_of(i), _CH)], sem_s[s])

    def chunk(i, carry):
        @pl.when(i % 2 == 0)
        def _():
            body(i, 0)

        @pl.when(i % 2 == 1)
        def _():
            body(i, 1)

        return carry

    lax.fori_loop(0, n_ch, chunk, 0)

    def drain(j, s):
        pltpu.make_async_copy(gsum[s],
                              gsum_hbm.at[pl.ds(loc_of(j), _CH)],
                              sem_s[s]).wait()

    last0 = n_ch - 2
    last1 = n_ch - 1

    @pl.when(last0 % 2 == 0)
    def _():
        drain(last0, 0)
        drain(last1, 1)

    @pl.when(last0 % 2 == 1)
    def _():
        drain(last0, 1)
        drain(last1, 0)


def _sc_gather(xCat, idxN, idxP, e0, n_edges):
    mesh = plsc.VectorSubcoreMesh(core_axis_name="c", subcore_axis_name="s")
    fn = functools.partial(
        pl.kernel,
        out_type=jax.ShapeDtypeStruct((n_edges, F), jnp.float32),
        mesh=mesh,
        scratch_types=[
            pltpu.VMEM((_CH,), jnp.int32),
            pltpu.VMEM((_CH,), jnp.int32),
            pltpu.VMEM((_CH,), jnp.int32),
            pltpu.VMEM((_CH,), jnp.int32),
            pltpu.VMEM((_CH, 2 * F), jnp.float32),
            pltpu.VMEM((_CH, 2 * F), jnp.float32),
            pltpu.VMEM((_CH, 2 * F), jnp.float32),
            pltpu.VMEM((_CH, 2 * F), jnp.float32),
            pltpu.VMEM((_CH, F), jnp.float32),
            pltpu.VMEM((_CH, F), jnp.float32),
            pltpu.SemaphoreType.DMA,
            pltpu.SemaphoreType.DMA,
            pltpu.SemaphoreType.DMA,
            pltpu.SemaphoreType.DMA,
            pltpu.SemaphoreType.DMA,
            pltpu.SemaphoreType.DMA,
        ],
    )(functools.partial(_gather_body, e0, n_edges))
    return fn(xCat, idxN, idxP)


_EDGE_BLK = 8000


def _edge_update_body(bond_ref, gsum_ref, wsum_ref, rker_ref, gker_ref,
                      bias_ref, scale_ref, shift_ref, gb0_ref, gb1_ref,
                      out_ref):
    bond = bond_ref[...]
    bondw = jnp.dot(bond, wsum_ref[...], preferred_element_type=jnp.float32)
    inner = jnp.dot(bond, rker_ref[...], preferred_element_type=jnp.float32) + gb1_ref[...]
    pre = gsum_ref[...] + bondw + bias_ref[...]
    a = jnp.maximum(pre, 0.0) * scale_ref[...] + shift_ref[...]
    mx = jnp.dot(a, gker_ref[...], preferred_element_type=jnp.float32) + gb0_ref[...]
    x_z = mx[:, :F]
    x_r = mx[:, F:2 * F]
    x_h = mx[:, 2 * F:]
    r_z = inner[:, :F]
    r_r = inner[:, F:2 * F]
    r_h = inner[:, 2 * F:]
    z = jax.nn.sigmoid(x_z + r_z)
    r = jax.nn.sigmoid(x_r + r_r)
    hh = jnp.tanh(x_h + r * r_h)
    out_ref[...] = z * bond + (1.0 - z) * hh


def _edge_update(bond_x, gsum, wsum, rker, gker, bias, scale, shift, gb0, gb1,
                 blk0):
    n_blk = gsum.shape[0] // _EDGE_BLK
    blk = lambda i: (i, 0)
    bblk = lambda i: (i + blk0, 0)
    full = lambda i: (0, 0)
    return pl.pallas_call(
        _edge_update_body,
        grid=(n_blk,),
        in_specs=[
            pl.BlockSpec((_EDGE_BLK, F), bblk),
            pl.BlockSpec((_EDGE_BLK, F), blk),
            pl.BlockSpec((F, F), full),
            pl.BlockSpec((F, 3 * F), full),
            pl.BlockSpec((F, 3 * F), full),
            pl.BlockSpec((1, F), full),
            pl.BlockSpec((1, F), full),
            pl.BlockSpec((1, F), full),
            pl.BlockSpec((1, 3 * F), full),
            pl.BlockSpec((1, 3 * F), full),
        ],
        out_specs=pl.BlockSpec((_EDGE_BLK, F), blk),
        out_shape=jax.ShapeDtypeStruct(gsum.shape, jnp.float32),
    )(bond_x, gsum, wsum, rker, gker, bias, scale, shift, gb0, gb1)


def kernel(x, bond_x, pairsNext, pairsPrev, wNext, wPrev, bNext, bPrev,
           gamma, beta, moving_mean, moving_var,
           gru_kernel, gru_recurrent_kernel, gru_bias):
    idxN = pairsNext[:, 1].astype(jnp.int32)
    idxP = pairsPrev[:, 1].astype(jnp.int32)

    scale = (gamma * lax.rsqrt(moving_var + 1e-3)).reshape(1, F)
    shift = (beta - moving_mean * gamma * lax.rsqrt(moving_var + 1e-3)).reshape(1, F)
    bias = bNext + bPrev
    wsum = wNext + wPrev
    gb0 = gru_bias[0].reshape(1, 3 * F)
    gb1 = gru_bias[1].reshape(1, 3 * F)

    wcat = jnp.concatenate([wNext, wPrev], axis=1)
    xCat = _node_transform(x, wcat)
    half = N_EDGES // 2
    gsum_a = _sc_gather(xCat, idxN, idxP, 0, half)
    gsum_b = _sc_gather(xCat, idxN, idxP, half, half)
    out_a = _edge_update(bond_x, gsum_a, wsum, gru_recurrent_kernel,
                         gru_kernel, bias, scale, shift, gb0, gb1, 0)
    out_b = _edge_update(bond_x, gsum_b, wsum, gru_recurrent_kernel,
                         gru_kernel, bias, scale, shift, gb0, gb1,
                         half // _EDGE_BLK)
    return jnp.concatenate([out_a, out_b], axis=0)


# speedup vs baseline: 1.0909x; 1.0909x over previous
import functools

import jax
import jax.numpy as jnp
from jax import lax
from jax.experimental import pallas as pl
from jax.experimental.pallas import tpu as pltpu
from jax.experimental.pallas import tpu_sc as plsc

N_NODES = 50000
N_EDGES = 800000
F = 64

_NODE_BLK = 2000


def _node_transform_body(x_ref, wcat_ref, xcat_ref):
    xcat_ref[...] = jnp.dot(x_ref[...], wcat_ref[...],
                            preferred_element_type=jnp.float32)


def _node_transform(x, wcat):
    n_blk = N_NODES // _NODE_BLK
    return pl.pallas_call(
        _node_transform_body,
        grid=(n_blk,),
        in_specs=[
            pl.BlockSpec((_NODE_BLK, F), lambda i: (i, 0)),
            pl.BlockSpec((F, 2 * F), lambda i: (0, 0)),
        ],
        out_specs=pl.BlockSpec((_NODE_BLK, 2 * F), lambda i: (i, 0)),
        out_shape=jax.ShapeDtypeStruct((N_NODES, 2 * F), jnp.float32),
    )(x, wcat)


_NW = 32
_PER_W = N_EDGES // _NW
_CH = 128


def _gather_body(e0, n_edges, xcat_hbm, idxn_hbm, idxp_hbm, gsum_hbm,
                 idxn_v0, idxn_v1, idxp_v0, idxp_v1,
                 rows_n0, rows_n1, rows_p0, rows_p1, gsum_v0, gsum_v1,
                 sem_i0, sem_i1, sem_g0, sem_g1, sem_s0, sem_s1):
    wid = lax.axis_index("s") * 2 + lax.axis_index("c")
    nchunks = n_edges // _CH
    nch_base = nchunks // _NW
    nch_rem = nchunks - nch_base * _NW
    n_ch = nch_base + jnp.where(wid < nch_rem, 1, 0)
    loc_of = lambda j: (wid + _NW * j) * _CH
    off_of = lambda j: e0 + loc_of(j)
    idxn = (idxn_v0, idxn_v1)
    idxp = (idxp_v0, idxp_v1)
    rows_n = (rows_n0, rows_n1)
    rows_p = (rows_p0, rows_p1)
    sem_i = (sem_i0, sem_i1)
    sem_g = (sem_g0, sem_g1)
    gsum = (gsum_v0, gsum_v1)
    sem_s = (sem_s0, sem_s1)

    def issue_idx(i, s):
        off = off_of(i)
        pltpu.async_copy(idxn_hbm.at[pl.ds(off, _CH)], idxn[s], sem_i[s])
        pltpu.async_copy(idxp_hbm.at[pl.ds(off, _CH)], idxp[s], sem_i[s])

    def wait_idx(i, s):
        off = off_of(i)
        pltpu.make_async_copy(idxn_hbm.at[pl.ds(off, _CH)], idxn[s], sem_i[s]).wait()
        pltpu.make_async_copy(idxp_hbm.at[pl.ds(off, _CH)], idxp[s], sem_i[s]).wait()

    def issue_gather(s):
        pltpu.async_copy(xcat_hbm.at[idxn[s]], rows_n[s], sem_g[s])
        pltpu.async_copy(xcat_hbm.at[idxp[s]], rows_p[s], sem_g[s])

    def wait_gather(s):
        pltpu.make_async_copy(xcat_hbm.at[idxn[s]], rows_n[s], sem_g[s]).wait()
        pltpu.make_async_copy(xcat_hbm.at[idxp[s]], rows_p[s], sem_g[s]).wait()

    issue_idx(0, 0)
    wait_idx(0, 0)
    issue_gather(0)
    issue_idx(1, 1)

    def body(i, s):
        q = 1 - s

        @pl.when(i + 1 < n_ch)
        def _():
            wait_idx(i + 1, q)
            issue_gather(q)

        wait_gather(s)

        @pl.when(i + 2 < n_ch)
        def _():
            issue_idx(i + 2, s)

        @pl.when(i >= 2)
        def _():
            off2 = loc_of(i - 2)
            pltpu.make_async_copy(gsum[s], gsum_hbm.at[pl.ds(off2, _CH)],
                                  sem_s[s]).wait()

        def row(r, c):
            for k in range(F // 16):
                a = rows_n[s][r, pl.ds(16 * k, 16)]
                b = rows_p[s][r, pl.ds(F + 16 * k, 16)]
                gsum[s][r, pl.ds(16 * k, 16)] = a + b
            return c

        lax.fori_loop(0, _CH, row, 0)
        pltpu.async_copy(gsum[s], gsum_hbm.at[pl.ds(loc_of(i), _CH)], sem_s[s])

    def chunk(i, carry):
        @pl.when(i % 2 == 0)
        def _():
            body(i, 0)

        @pl.when(i % 2 == 1)
        def _():
            body(i, 1)

        return carry

    lax.fori_loop(0, n_ch, chunk, 0)

    def drain(j, s):
        pltpu.make_async_copy(gsum[s],
                              gsum_hbm.at[pl.ds(loc_of(j), _CH)],
                              sem_s[s]).wait()

    last0 = n_ch - 2
    last1 = n_ch - 1

    @pl.when(last0 % 2 == 0)
    def _():
        drain(last0, 0)
        drain(last1, 1)

    @pl.when(last0 % 2 == 1)
    def _():
        drain(last0, 1)
        drain(last1, 0)


def _sc_gather(xCat, idxN, idxP, e0, n_edges):
    mesh = plsc.VectorSubcoreMesh(core_axis_name="c", subcore_axis_name="s")
    fn = functools.partial(
        pl.kernel,
        out_type=jax.ShapeDtypeStruct((n_edges, F), jnp.float32),
        mesh=mesh,
        scratch_types=[
            pltpu.VMEM((_CH,), jnp.int32),
            pltpu.VMEM((_CH,), jnp.int32),
            pltpu.VMEM((_CH,), jnp.int32),
            pltpu.VMEM((_CH,), jnp.int32),
            pltpu.VMEM((_CH, 2 * F), jnp.float32),
            pltpu.VMEM((_CH, 2 * F), jnp.float32),
            pltpu.VMEM((_CH, 2 * F), jnp.float32),
            pltpu.VMEM((_CH, 2 * F), jnp.float32),
            pltpu.VMEM((_CH, F), jnp.float32),
            pltpu.VMEM((_CH, F), jnp.float32),
            pltpu.SemaphoreType.DMA,
            pltpu.SemaphoreType.DMA,
            pltpu.SemaphoreType.DMA,
            pltpu.SemaphoreType.DMA,
            pltpu.SemaphoreType.DMA,
            pltpu.SemaphoreType.DMA,
        ],
    )(functools.partial(_gather_body, e0, n_edges))
    return fn(xCat, idxN, idxP)


_EDGE_BLK = 8000


def _edge_update_body(bond_ref, gsum_ref, wsum_ref, rker_ref, gker_ref,
                      bias_ref, scale_ref, shift_ref, gb0_ref, gb1_ref,
                      out_ref):
    bond = bond_ref[...]
    bondw = jnp.dot(bond, wsum_ref[...], preferred_element_type=jnp.float32)
    inner = jnp.dot(bond, rker_ref[...], preferred_element_type=jnp.float32) + gb1_ref[...]
    pre = gsum_ref[...] + bondw + bias_ref[...]
    a = jnp.maximum(pre, 0.0) * scale_ref[...] + shift_ref[...]
    mx = jnp.dot(a, gker_ref[...], preferred_element_type=jnp.float32) + gb0_ref[...]
    x_z = mx[:, :F]
    x_r = mx[:, F:2 * F]
    x_h = mx[:, 2 * F:]
    r_z = inner[:, :F]
    r_r = inner[:, F:2 * F]
    r_h = inner[:, 2 * F:]
    z = jax.nn.sigmoid(x_z + r_z)
    r = jax.nn.sigmoid(x_r + r_r)
    hh = jnp.tanh(x_h + r * r_h)
    out_ref[...] = z * bond + (1.0 - z) * hh


def _edge_update(bond_x, gsum, wsum, rker, gker, bias, scale, shift, gb0, gb1,
                 blk0):
    n_blk = gsum.shape[0] // _EDGE_BLK
    blk = lambda i: (i, 0)
    bblk = lambda i: (i + blk0, 0)
    full = lambda i: (0, 0)
    return pl.pallas_call(
        _edge_update_body,
        grid=(n_blk,),
        in_specs=[
            pl.BlockSpec((_EDGE_BLK, F), bblk),
            pl.BlockSpec((_EDGE_BLK, F), blk),
            pl.BlockSpec((F, F), full),
            pl.BlockSpec((F, 3 * F), full),
            pl.BlockSpec((F, 3 * F), full),
            pl.BlockSpec((1, F), full),
            pl.BlockSpec((1, F), full),
            pl.BlockSpec((1, F), full),
            pl.BlockSpec((1, 3 * F), full),
            pl.BlockSpec((1, 3 * F), full),
        ],
        out_specs=pl.BlockSpec((_EDGE_BLK, F), blk),
        out_shape=jax.ShapeDtypeStruct(gsum.shape, jnp.float32),
    )(bond_x, gsum, wsum, rker, gker, bias, scale, shift, gb0, gb1)


def kernel(x, bond_x, pairsNext, pairsPrev, wNext, wPrev, bNext, bPrev,
           gamma, beta, moving_mean, moving_var,
           gru_kernel, gru_recurrent_kernel, gru_bias):
    idxN = pairsNext[:, 1].astype(jnp.int32)
    idxP = pairsPrev[:, 1].astype(jnp.int32)

    scale = (gamma * lax.rsqrt(moving_var + 1e-3)).reshape(1, F)
    shift = (beta - moving_mean * gamma * lax.rsqrt(moving_var + 1e-3)).reshape(1, F)
    bias = bNext + bPrev
    wsum = wNext + wPrev
    gb0 = gru_bias[0].reshape(1, 3 * F)
    gb1 = gru_bias[1].reshape(1, 3 * F)

    wcat = jnp.concatenate([wNext, wPrev], axis=1)
    xCat = _node_transform(x, wcat)
    gsum = _sc_gather(xCat, idxN, idxP, 0, N_EDGES)
    return _edge_update(bond_x, gsum, wsum, gru_recurrent_kernel,
                        gru_kernel, bias, scale, shift, gb0, gb1, 0)


# speedup vs baseline: 1.1187x; 1.0256x over previous
import functools

import jax
import jax.numpy as jnp
from jax import lax
from jax.experimental import pallas as pl
from jax.experimental.pallas import tpu as pltpu
from jax.experimental.pallas import tpu_sc as plsc

N_NODES = 50000
N_EDGES = 800000
F = 64

_NODE_BLK = 2000


def _node_transform_body(x_ref, wcat_ref, xcat_ref):
    xcat_ref[...] = jnp.dot(x_ref[...], wcat_ref[...],
                            preferred_element_type=jnp.float32)


def _node_transform(x, wcat):
    n_blk = N_NODES // _NODE_BLK
    return pl.pallas_call(
        _node_transform_body,
        grid=(n_blk,),
        in_specs=[
            pl.BlockSpec((_NODE_BLK, F), lambda i: (i, 0)),
            pl.BlockSpec((F, 2 * F), lambda i: (0, 0)),
        ],
        out_specs=pl.BlockSpec((_NODE_BLK, 2 * F), lambda i: (i, 0)),
        out_shape=jax.ShapeDtypeStruct((N_NODES, 2 * F), jnp.float32),
    )(x, wcat)


_NW = 32
_PER_W = N_EDGES // _NW
_CH = 128


def _gather_body(e0, n_edges, xcat_hbm, idxn_hbm, idxp_hbm, gsum_hbm,
                 idxn_v0, idxn_v1, idxp_v0, idxp_v1,
                 rows_n0, rows_n1, rows_p0, rows_p1, gsum_v0, gsum_v1,
                 sem_i0, sem_i1, sem_g0, sem_g1, sem_s0, sem_s1):
    wid = lax.axis_index("s") * 2 + lax.axis_index("c")
    nchunks = n_edges // _CH
    nch_base = nchunks // _NW
    nch_rem = nchunks - nch_base * _NW
    n_ch = nch_base + jnp.where(wid < nch_rem, 1, 0)
    loc_of = lambda j: (wid + _NW * j) * _CH
    off_of = lambda j: e0 + loc_of(j)
    idxn = (idxn_v0, idxn_v1)
    idxp = (idxp_v0, idxp_v1)
    rows_n = (rows_n0, rows_n1)
    rows_p = (rows_p0, rows_p1)
    sem_i = (sem_i0, sem_i1)
    sem_g = (sem_g0, sem_g1)
    gsum = (gsum_v0, gsum_v1)
    sem_s = (sem_s0, sem_s1)

    def issue_idx(i, s):
        off = off_of(i)
        pltpu.async_copy(idxn_hbm.at[pl.ds(off, _CH)], idxn[s], sem_i[s])
        pltpu.async_copy(idxp_hbm.at[pl.ds(off, _CH)], idxp[s], sem_i[s])

    def wait_idx(i, s):
        off = off_of(i)
        pltpu.make_async_copy(idxn_hbm.at[pl.ds(off, _CH)], idxn[s], sem_i[s]).wait()
        pltpu.make_async_copy(idxp_hbm.at[pl.ds(off, _CH)], idxp[s], sem_i[s]).wait()

    def issue_gather(s):
        pltpu.async_copy(xcat_hbm.at[idxn[s]], rows_n[s], sem_g[s])
        pltpu.async_copy(xcat_hbm.at[idxp[s]], rows_p[s], sem_g[s])

    def wait_gather(s):
        pltpu.make_async_copy(xcat_hbm.at[idxn[s]], rows_n[s], sem_g[s]).wait()
        pltpu.make_async_copy(xcat_hbm.at[idxp[s]], rows_p[s], sem_g[s]).wait()

    issue_idx(0, 0)
    wait_idx(0, 0)
    issue_gather(0)
    issue_idx(1, 1)

    def body(i, s):
        q = 1 - s

        @pl.when(i + 1 < n_ch)
        def _():
            wait_idx(i + 1, q)
            issue_gather(q)

        wait_gather(s)

        @pl.when(i + 2 < n_ch)
        def _():
            issue_idx(i + 2, s)

        @pl.when(i >= 2)
        def _():
            off2 = loc_of(i - 2)
            pltpu.make_async_copy(gsum[s], gsum_hbm.at[pl.ds(off2, _CH)],
                                  sem_s[s]).wait()

        def row(r, c):
            for k in range(F // 16):
                a = rows_n[s][r, pl.ds(16 * k, 16)]
                b = rows_p[s][r, pl.ds(F + 16 * k, 16)]
                gsum[s][r, pl.ds(16 * k, 16)] = a + b
            return c

        lax.fori_loop(0, _CH, row, 0)
        pltpu.async_copy(gsum[s], gsum_hbm.at[pl.ds(loc_of(i), _CH)], sem_s[s])

    def chunk(i, carry):
        @pl.when(i % 2 == 0)
        def _():
            body(i, 0)

        @pl.when(i % 2 == 1)
        def _():
            body(i, 1)

        return carry

    lax.fori_loop(0, n_ch, chunk, 0)

    def drain(j, s):
        pltpu.make_async_copy(gsum[s],
                              gsum_hbm.at[pl.ds(loc_of(j), _CH)],
                              sem_s[s]).wait()

    last0 = n_ch - 2
    last1 = n_ch - 1

    @pl.when(last0 % 2 == 0)
    def _():
        drain(last0, 0)
        drain(last1, 1)

    @pl.when(last0 % 2 == 1)
    def _():
        drain(last0, 1)
        drain(last1, 0)


def _sc_gather(xCat, idxN, idxP, e0, n_edges):
    mesh = plsc.VectorSubcoreMesh(core_axis_name="c", subcore_axis_name="s")
    fn = functools.partial(
        pl.kernel,
        out_type=jax.ShapeDtypeStruct((n_edges, F), jnp.float32),
        mesh=mesh,
        scratch_types=[
            pltpu.VMEM((_CH,), jnp.int32),
            pltpu.VMEM((_CH,), jnp.int32),
            pltpu.VMEM((_CH,), jnp.int32),
            pltpu.VMEM((_CH,), jnp.int32),
            pltpu.VMEM((_CH, 2 * F), jnp.float32),
            pltpu.VMEM((_CH, 2 * F), jnp.float32),
            pltpu.VMEM((_CH, 2 * F), jnp.float32),
            pltpu.VMEM((_CH, 2 * F), jnp.float32),
            pltpu.VMEM((_CH, F), jnp.float32),
            pltpu.VMEM((_CH, F), jnp.float32),
            pltpu.SemaphoreType.DMA,
            pltpu.SemaphoreType.DMA,
            pltpu.SemaphoreType.DMA,
            pltpu.SemaphoreType.DMA,
            pltpu.SemaphoreType.DMA,
            pltpu.SemaphoreType.DMA,
        ],
    )(functools.partial(_gather_body, e0, n_edges))
    return fn(xCat, idxN, idxP)


_EDGE_BLK = 16000


def _edge_update_body(bond_ref, gsum_ref, wsum_ref, rker_ref, gker_ref,
                      bias_ref, scale_ref, shift_ref, gb0_ref, gb1_ref,
                      out_ref):
    bond = bond_ref[...]
    bondw = jnp.dot(bond, wsum_ref[...], preferred_element_type=jnp.float32)
    inner = jnp.dot(bond, rker_ref[...], preferred_element_type=jnp.float32) + gb1_ref[...]
    pre = gsum_ref[...] + bondw + bias_ref[...]
    a = jnp.maximum(pre, 0.0) * scale_ref[...] + shift_ref[...]
    mx = jnp.dot(a, gker_ref[...], preferred_element_type=jnp.float32) + gb0_ref[...]
    x_z = mx[:, :F]
    x_r = mx[:, F:2 * F]
    x_h = mx[:, 2 * F:]
    r_z = inner[:, :F]
    r_r = inner[:, F:2 * F]
    r_h = inner[:, 2 * F:]
    z = jax.nn.sigmoid(x_z + r_z)
    r = jax.nn.sigmoid(x_r + r_r)
    hh = jnp.tanh(x_h + r * r_h)
    out_ref[...] = z * bond + (1.0 - z) * hh


def _edge_update(bond_x, gsum, wsum, rker, gker, bias, scale, shift, gb0, gb1,
                 blk0):
    n_blk = gsum.shape[0] // _EDGE_BLK
    blk = lambda i: (i, 0)
    bblk = lambda i: (i + blk0, 0)
    full = lambda i: (0, 0)
    return pl.pallas_call(
        _edge_update_body,
        grid=(n_blk,),
        in_specs=[
            pl.BlockSpec((_EDGE_BLK, F), bblk),
            pl.BlockSpec((_EDGE_BLK, F), blk),
            pl.BlockSpec((F, F), full),
            pl.BlockSpec((F, 3 * F), full),
            pl.BlockSpec((F, 3 * F), full),
            pl.BlockSpec((1, F), full),
            pl.BlockSpec((1, F), full),
            pl.BlockSpec((1, F), full),
            pl.BlockSpec((1, 3 * F), full),
            pl.BlockSpec((1, 3 * F), full),
        ],
        out_specs=pl.BlockSpec((_EDGE_BLK, F), blk),
        out_shape=jax.ShapeDtypeStruct(gsum.shape, jnp.float32),
    )(bond_x, gsum, wsum, rker, gker, bias, scale, shift, gb0, gb1)


def kernel(x, bond_x, pairsNext, pairsPrev, wNext, wPrev, bNext, bPrev,
           gamma, beta, moving_mean, moving_var,
           gru_kernel, gru_recurrent_kernel, gru_bias):
    idxN = pairsNext[:, 1].astype(jnp.int32)
    idxP = pairsPrev[:, 1].astype(jnp.int32)

    scale = (gamma * lax.rsqrt(moving_var + 1e-3)).reshape(1, F)
    shift = (beta - moving_mean * gamma * lax.rsqrt(moving_var + 1e-3)).reshape(1, F)
    bias = bNext + bPrev
    wsum = wNext + wPrev
    gb0 = gru_bias[0].reshape(1, 3 * F)
    gb1 = gru_bias[1].reshape(1, 3 * F)

    wcat = jnp.concatenate([wNext, wPrev], axis=1)
    xCat = _node_transform(x, wcat)
    gsum = _sc_gather(xCat, idxN, idxP, 0, N_EDGES)
    return _edge_update(bond_x, gsum, wsum, gru_recurrent_kernel,
                        gru_kernel, bias, scale, shift, gb0, gb1, 0)


# speedup vs baseline: 1.1210x; 1.0020x over previous
import functools

import jax
import jax.numpy as jnp
from jax import lax
from jax.experimental import pallas as pl
from jax.experimental.pallas import tpu as pltpu
from jax.experimental.pallas import tpu_sc as plsc

N_NODES = 50000
N_EDGES = 800000
F = 64

_NODE_BLK = 2000


def _node_transform_body(x_ref, wcat_ref, xcat_ref):
    xcat_ref[...] = jnp.dot(x_ref[...], wcat_ref[...],
                            preferred_element_type=jnp.float32)


def _node_transform(x, wcat):
    n_blk = N_NODES // _NODE_BLK
    return pl.pallas_call(
        _node_transform_body,
        grid=(n_blk,),
        in_specs=[
            pl.BlockSpec((_NODE_BLK, F), lambda i: (i, 0)),
            pl.BlockSpec((F, 2 * F), lambda i: (0, 0)),
        ],
        out_specs=pl.BlockSpec((_NODE_BLK, 2 * F), lambda i: (i, 0)),
        out_shape=jax.ShapeDtypeStruct((N_NODES, 2 * F), jnp.float32),
    )(x, wcat)


_NW = 32
_PER_W = N_EDGES // _NW
_CH = 80


def _gather_body(e0, n_edges, xcat_hbm, idxn_hbm, idxp_hbm, gsum_hbm,
                 idxn_v0, idxn_v1, idxn_v2, idxp_v0, idxp_v1, idxp_v2,
                 rows_n0, rows_n1, rows_n2, rows_p0, rows_p1, rows_p2,
                 gsum_v0, gsum_v1, gsum_v2,
                 sem_i0, sem_i1, sem_i2, sem_g0, sem_g1, sem_g2,
                 sem_s0, sem_s1, sem_s2):
    wid = lax.axis_index("s") * 2 + lax.axis_index("c")
    nchunks = n_edges // _CH
    nch_base = nchunks // _NW
    nch_rem = nchunks - nch_base * _NW
    n_ch = nch_base + jnp.where(wid < nch_rem, 1, 0)
    loc_of = lambda j: (wid + _NW * j) * _CH
    off_of = lambda j: e0 + loc_of(j)
    idxn = (idxn_v0, idxn_v1, idxn_v2)
    idxp = (idxp_v0, idxp_v1, idxp_v2)
    rows_n = (rows_n0, rows_n1, rows_n2)
    rows_p = (rows_p0, rows_p1, rows_p2)
    gsum = (gsum_v0, gsum_v1, gsum_v2)
    sem_i = (sem_i0, sem_i1, sem_i2)
    sem_g = (sem_g0, sem_g1, sem_g2)
    sem_s = (sem_s0, sem_s1, sem_s2)

    def issue_idx(i, s):
        off = off_of(i)
        pltpu.async_copy(idxn_hbm.at[pl.ds(off, _CH)], idxn[s], sem_i[s])
        pltpu.async_copy(idxp_hbm.at[pl.ds(off, _CH)], idxp[s], sem_i[s])

    def wait_idx(i, s):
        off = off_of(i)
        pltpu.make_async_copy(idxn_hbm.at[pl.ds(off, _CH)], idxn[s], sem_i[s]).wait()
        pltpu.make_async_copy(idxp_hbm.at[pl.ds(off, _CH)], idxp[s], sem_i[s]).wait()

    def issue_gather(s):
        pltpu.async_copy(xcat_hbm.at[idxn[s]], rows_n[s], sem_g[s])
        pltpu.async_copy(xcat_hbm.at[idxp[s]], rows_p[s], sem_g[s])

    def wait_gather(s):
        pltpu.make_async_copy(xcat_hbm.at[idxn[s]], rows_n[s], sem_g[s]).wait()
        pltpu.make_async_copy(xcat_hbm.at[idxp[s]], rows_p[s], sem_g[s]).wait()

    def wait_store(j, s):
        pltpu.make_async_copy(gsum[s], gsum_hbm.at[pl.ds(loc_of(j), _CH)],
                              sem_s[s]).wait()

    issue_idx(0, 0)
    issue_idx(1, 1)
    issue_idx(2, 2)
    wait_idx(0, 0)
    issue_gather(0)
    wait_idx(1, 1)
    issue_gather(1)

    def body(i, s):
        s2 = (s + 2) % 3

        @pl.when(i + 2 < n_ch)
        def _():
            wait_idx(i + 2, s2)
            issue_gather(s2)

        wait_gather(s)

        @pl.when(i + 3 < n_ch)
        def _():
            issue_idx(i + 3, s)

        @pl.when(i >= 3)
        def _():
            wait_store(i - 3, s)

        def row(r, c):
            for k in range(F // 16):
                a = rows_n[s][r, pl.ds(16 * k, 16)]
                b = rows_p[s][r, pl.ds(F + 16 * k, 16)]
                gsum[s][r, pl.ds(16 * k, 16)] = a + b
            return c

        lax.fori_loop(0, _CH, row, 0)
        pltpu.async_copy(gsum[s], gsum_hbm.at[pl.ds(loc_of(i), _CH)], sem_s[s])

    def chunk(i, carry):
        for s in range(3):
            @pl.when(i % 3 == s)
            def _(s=s):
                body(i, s)
        return carry

    lax.fori_loop(0, n_ch, chunk, 0)

    for d in (3, 2, 1):
        j = n_ch - d
        for s in range(3):
            @pl.when(j % 3 == s)
            def _(j=j, s=s):
                wait_store(j, s)


def _sc_gather(xCat, idxN, idxP, e0, n_edges):
    mesh = plsc.VectorSubcoreMesh(core_axis_name="c", subcore_axis_name="s")
    fn = functools.partial(
        pl.kernel,
        out_type=jax.ShapeDtypeStruct((n_edges, F), jnp.float32),
        mesh=mesh,
        scratch_types=[
            pltpu.VMEM((_CH,), jnp.int32),
            pltpu.VMEM((_CH,), jnp.int32),
            pltpu.VMEM((_CH,), jnp.int32),
            pltpu.VMEM((_CH,), jnp.int32),
            pltpu.VMEM((_CH,), jnp.int32),
            pltpu.VMEM((_CH,), jnp.int32),
            pltpu.VMEM((_CH, 2 * F), jnp.float32),
            pltpu.VMEM((_CH, 2 * F), jnp.float32),
            pltpu.VMEM((_CH, 2 * F), jnp.float32),
            pltpu.VMEM((_CH, 2 * F), jnp.float32),
            pltpu.VMEM((_CH, 2 * F), jnp.float32),
            pltpu.VMEM((_CH, 2 * F), jnp.float32),
            pltpu.VMEM((_CH, F), jnp.float32),
            pltpu.VMEM((_CH, F), jnp.float32),
            pltpu.VMEM((_CH, F), jnp.float32),
        ] + [pltpu.SemaphoreType.DMA] * 9,
    )(functools.partial(_gather_body, e0, n_edges))
    return fn(xCat, idxN, idxP)


_EDGE_BLK = 16000


def _edge_update_body(bond_ref, gsum_ref, wsum_ref, rker_ref, gker_ref,
                      bias_ref, scale_ref, shift_ref, gb0_ref, gb1_ref,
                      out_ref):
    bond = bond_ref[...]
    bondw = jnp.dot(bond, wsum_ref[...], preferred_element_type=jnp.float32)
    inner = jnp.dot(bond, rker_ref[...], preferred_element_type=jnp.float32) + gb1_ref[...]
    pre = gsum_ref[...] + bondw + bias_ref[...]
    a = jnp.maximum(pre, 0.0) * scale_ref[...] + shift_ref[...]
    mx = jnp.dot(a, gker_ref[...], preferred_element_type=jnp.float32) + gb0_ref[...]
    x_z = mx[:, :F]
    x_r = mx[:, F:2 * F]
    x_h = mx[:, 2 * F:]
    r_z = inner[:, :F]
    r_r = inner[:, F:2 * F]
    r_h = inner[:, 2 * F:]
    z = jax.nn.sigmoid(x_z + r_z)
    r = jax.nn.sigmoid(x_r + r_r)
    hh = jnp.tanh(x_h + r * r_h)
    out_ref[...] = z * bond + (1.0 - z) * hh


def _edge_update(bond_x, gsum, wsum, rker, gker, bias, scale, shift, gb0, gb1,
                 blk0):
    n_blk = gsum.shape[0] // _EDGE_BLK
    blk = lambda i: (i, 0)
    bblk = lambda i: (i + blk0, 0)
    full = lambda i: (0, 0)
    return pl.pallas_call(
        _edge_update_body,
        grid=(n_blk,),
        in_specs=[
            pl.BlockSpec((_EDGE_BLK, F), bblk),
            pl.BlockSpec((_EDGE_BLK, F), blk),
            pl.BlockSpec((F, F), full),
            pl.BlockSpec((F, 3 * F), full),
            pl.BlockSpec((F, 3 * F), full),
            pl.BlockSpec((1, F), full),
            pl.BlockSpec((1, F), full),
            pl.BlockSpec((1, F), full),
            pl.BlockSpec((1, 3 * F), full),
            pl.BlockSpec((1, 3 * F), full),
        ],
        out_specs=pl.BlockSpec((_EDGE_BLK, F), blk),
        out_shape=jax.ShapeDtypeStruct(gsum.shape, jnp.float32),
    )(bond_x, gsum, wsum, rker, gker, bias, scale, shift, gb0, gb1)


def kernel(x, bond_x, pairsNext, pairsPrev, wNext, wPrev, bNext, bPrev,
           gamma, beta, moving_mean, moving_var,
           gru_kernel, gru_recurrent_kernel, gru_bias):
    idxN = pairsNext[:, 1].astype(jnp.int32)
    idxP = pairsPrev[:, 1].astype(jnp.int32)

    scale = (gamma * lax.rsqrt(moving_var + 1e-3)).reshape(1, F)
    shift = (beta - moving_mean * gamma * lax.rsqrt(moving_var + 1e-3)).reshape(1, F)
    bias = bNext + bPrev
    wsum = wNext + wPrev
    gb0 = gru_bias[0].reshape(1, 3 * F)
    gb1 = gru_bias[1].reshape(1, 3 * F)

    wcat = jnp.concatenate([wNext, wPrev], axis=1)
    xCat = _node_transform(x, wcat)
    gsum = _sc_gather(xCat, idxN, idxP, 0, N_EDGES)
    return _edge_update(bond_x, gsum, wsum, gru_recurrent_kernel,
                        gru_kernel, bias, scale, shift, gb0, gb1, 0)
